# baseline (speedup 1.0000x reference)
import functools

import jax
import jax.numpy as jnp
from jax import lax
from jax.experimental import pallas as pl
from jax.experimental.pallas import tpu as pltpu
from jax.experimental.pallas import tpu_sc as plsc

_INTERPRET = False

NC, NS = 2, 16
NW = NC * NS
CHUNK_G = 96
NBUF_G = 4
CHUNK_S = 128
LAT = 128
BR = 1024


def _ln_f(x):
    m = jnp.mean(x, axis=-1, keepdims=True)
    xc = x - m
    v = jnp.mean(xc * xc, axis=-1, keepdims=True)
    return xc * lax.rsqrt(v + 1e-5)


def _relu(x):
    return jnp.maximum(x, 0.0)


def _dot(a, b):
    return jnp.dot(a, b, preferred_element_type=jnp.float32)


def _rb(d):
    return pl.BlockSpec((BR, d), lambda i: (i, 0))


def _full(shape):
    return pl.BlockSpec(shape, lambda i: tuple(0 for _ in shape))


def _sc_gather2(table_s, table_r, sidx3, ridx3, pos_table=None):
    k_chunks = sidx3.shape[1]
    d = table_s.shape[1]
    e_pad = NW * k_chunks * CHUNK_G
    assert k_chunks % NBUF_G == 0
    mesh = plsc.VectorSubcoreMesh(core_axis_name="c", subcore_axis_name="s")
    dp = 0 if pos_table is None else pos_table.shape[1]
    out_type = [jax.ShapeDtypeStruct((e_pad, d), jnp.float32)] * 2
    pos_scratch = []
    pos_args = ()
    if pos_table is not None:
        out_type += [jax.ShapeDtypeStruct((e_pad, dp), jnp.float32)] * 2
        pos_scratch = [pltpu.VMEM((CHUNK_G, dp), jnp.float32)] * (2 * NBUF_G)
        pos_args = (pos_table,)

    @functools.partial(
        pl.kernel,
        out_type=out_type,
        mesh=mesh,
        compiler_params=pltpu.CompilerParams(
            use_tc_tiling_on_sc=False,
            disable_bounds_checks=True,
            disable_semaphore_checks=True),
        scratch_types=[
            pltpu.VMEM((k_chunks, CHUNK_G), jnp.int32),
            pltpu.VMEM((k_chunks, CHUNK_G), jnp.int32),
        ] + [pltpu.VMEM((CHUNK_G, d), jnp.float32)] * (2 * NBUF_G)
          + pos_scratch + [
            pltpu.SemaphoreType.DMA,
            pltpu.SemaphoreType.DMA,
        ],
    )
    def gather_kernel(ts, tr, *rest):
        if dp:
            (tp, si, ri, gs, gr, gsp, grp, siv, riv,
             s0, s1, s2, s3, r0, r1, r2, r3,
             p0, p1, p2, p3, q0, q1, q2, q3, gsem, wsem) = rest
        else:
            (si, ri, gs, gr, siv, riv,
             s0, s1, s2, s3, r0, r1, r2, r3, gsem, wsem) = rest
        w = lax.axis_index("s") * NC + lax.axis_index("c")
        pltpu.sync_copy(si.at[w], siv)
        pltpu.sync_copy(ri.at[w], riv)

        def run_phase(ta, tb, oa, ob, abuf, bbuf):
            def start(j, b):
                pltpu.async_copy(ta.at[siv.at[j]], abuf[b], gsem)
                pltpu.async_copy(tb.at[riv.at[j]], bbuf[b], gsem)

            def wait_gather(b):
                pltpu.make_async_copy(ta.at[siv.at[0]], abuf[b], gsem).wait()
                pltpu.make_async_copy(tb.at[riv.at[0]], bbuf[b], gsem).wait()

            def start_write(j, b):
                base = (w * k_chunks + j) * CHUNK_G
                pltpu.async_copy(abuf[b], oa.at[pl.ds(base, CHUNK_G)], wsem)
                pltpu.async_copy(bbuf[b], ob.at[pl.ds(base, CHUNK_G)], wsem)

            def wait_write(b):
                pltpu.make_async_copy(
                    abuf[b], oa.at[pl.ds(0, CHUNK_G)], wsem).wait()
                pltpu.make_async_copy(
                    bbuf[b], ob.at[pl.ds(0, CHUNK_G)], wsem).wait()

            for b in range(NBUF_G - 1):
                start(b, b)

            def body(g, carry):
                for b in range(NBUF_G):
                    j = g * NBUF_G + b
                    jn = j + NBUF_G - 1
                    bn = (b + NBUF_G - 1) % NBUF_G

                    @pl.when(jnp.logical_and(jn < k_chunks, j >= 1))
                    def _():
                        wait_write(bn)

                    @pl.when(jn < k_chunks)
                    def _():
                        start(jn, bn)

                    wait_gather(b)
                    start_write(j, b)
                return carry

            lax.fori_loop(0, k_chunks // NBUF_G, body, 0)
            for b in range(NBUF_G):
                wait_write(b)

        run_phase(ts, tr, gs, gr, [s0, s1, s2, s3], [r0, r1, r2, r3])
        if dp:
            run_phase(tp, tp, gsp, grp, [p0, p1, p2, p3], [q0, q1, q2, q3])

    return gather_kernel(table_s, table_r, *pos_args, sidx3, ridx3)


def _sc_scatter(vals, ridx3, n_acc):
    k_chunks = ridx3.shape[1]
    d = vals.shape[1]
    rows_tile = n_acc // NS
    mesh = plsc.VectorSubcoreMesh(core_axis_name="c", subcore_axis_name="s")
    zeros_tile = jnp.zeros((rows_tile, d), jnp.float32)

    @functools.partial(
        pl.kernel,
        out_type=jax.ShapeDtypeStruct((NC, n_acc, d), jnp.float32),
        mesh=mesh,
        compiler_params=pltpu.CompilerParams(
            disable_bounds_checks=True,
            disable_semaphore_checks=True),
        scratch_types=[
            pltpu.VMEM((k_chunks, CHUNK_S), jnp.int32),
            pltpu.VMEM((CHUNK_S, d), jnp.float32),
            pltpu.VMEM((CHUNK_S, d), jnp.float32),
            pltpu.VMEM_SHARED((n_acc, d), jnp.float32),
            pltpu.SemaphoreType.DMA,
        ],
    )
    def scatter_kernel(vals_r, ri, zrows, out, idxv, v0, v1, acc, lsem):
        c = lax.axis_index("c")
        s = lax.axis_index("s")
        w = s * NC + c
        pltpu.sync_copy(zrows, acc.at[pl.ds(s * rows_tile, rows_tile)])
        pltpu.sync_copy(ri.at[w], idxv)
        plsc.subcore_barrier()
        vbuf = [v0, v1]

        def startv(j, b):
            base = (w * k_chunks + j) * CHUNK_S
            pltpu.async_copy(vals_r.at[pl.ds(base, CHUNK_S)], vbuf[b], lsem)

        def waitv(b):
            pltpu.make_async_copy(vals_r.at[pl.ds(0, CHUNK_S)], vbuf[b],
                                  lsem).wait()

        startv(0, 0)

        def body(g, carry):
            for b in range(2):
                j = 2 * g + b

                @pl.when(j + 1 < k_chunks)
                def _():
                    startv(j + 1, 1 - b)

                waitv(b)
                pltpu.sync_copy(vbuf[b], acc.at[idxv.at[j]], add=True)
            return carry

        lax.fori_loop(0, k_chunks // 2, body, 0)
        plsc.subcore_barrier()
        pltpu.sync_copy(acc.at[pl.ds(s * rows_tile, rows_tile)],
                        out.at[c, pl.ds(s * rows_tile, rows_tile)])

    return scatter_kernel(vals, ridx3, zeros_tile)


def _node_enc_body(pp, ppp, nt, w1v, w1o, b1, w2, b2, w3, b3, ws0, wr0,
                   on, ots, otr):
    vel = pp[...] - ppp[...]
    oh = (lax.broadcasted_iota(jnp.int32, (vel.shape[0], 16), 1)
          == nt[...]).astype(jnp.float32)
    x = _relu(_dot(vel, w1v[...]) + _dot(oh, w1o[...]) + b1[...])
    x = _relu(_dot(x, w2[...]) + b2[...])
    y = _ln_f(_dot(x, w3[...]) + b3[...])
    on[...] = y
    ots[...] = _dot(y, ws0[...])
    otr[...] = _dot(y, wr0[...])


def _edge_enc_body(ps, pr, w1d, wnw, wnm, b1, w2, b2, w3, b3, out):
    dd = ps[...] - pr[...]
    dw = dd[:, 0:3]
    dm = dd[:, 3:5]
    nw = jnp.sqrt(jnp.sum(dw * dw, axis=-1, keepdims=True) + 1e-12)
    nm = jnp.sqrt(jnp.sum(dm * dm, axis=-1, keepdims=True) + 1e-12)
    x = _relu(_dot(dd, w1d[...]) + nw * wnw[...] + nm * wnm[...] + b1[...])
    x = _relu(_dot(x, w2[...]) + b2[...])
    out[...] = _ln_f(_dot(x, w3[...]) + b3[...])


def _edge_step_body(gs, gr, e, we, b1, w2, b2, w3, b3, out):
    er = e[...]
    x = _relu(gs[...] + gr[...] + _dot(er, we[...]) + b1[...])
    x = _relu(_dot(x, w2[...]) + b2[...])
    out[...] = _ln_f(_dot(x, w3[...]) + b3[...]) + er


def _node_step_body(n, pp, wn, wa, b1, w2, b2, w3, b3, wsn, wrn,
                    on, ots, otr):
    nr = n[...]
    p = pp[...]
    agg = p[0] + p[1]
    x = _relu(_dot(nr, wn[...]) + _dot(agg, wa[...]) + b1[...])
    x = _relu(_dot(x, w2[...]) + b2[...])
    y = _ln_f(_dot(x, w3[...]) + b3[...]) + nr
    on[...] = y
    ots[...] = _dot(y, wsn[...])
    otr[...] = _dot(y, wrn[...])


def _node_step_last_body(n, pp, wn, wa, b1, w2, b2, w3, b3, on):
    nr = n[...]
    p = pp[...]
    agg = p[0] + p[1]
    x = _relu(_dot(nr, wn[...]) + _dot(agg, wa[...]) + b1[...])
    x = _relu(_dot(x, w2[...]) + b2[...])
    on[...] = _ln_f(_dot(x, w3[...]) + b3[...]) + nr


def _dec_body(n, pp, ppp, w1, b1, w2, b2, w3, b3, out):
    x = _relu(_dot(n[...], w1[...]) + b1[...])
    x = _relu(_dot(x, w2[...]) + b2[...])
    x = _dot(x, w3[...]) + b3[...]
    out[...] = x + 2.0 * pp[...] - ppp[...]


def _tc(body, out_shapes, grid, in_specs, out_specs):
    return pl.pallas_call(
        body, out_shape=out_shapes, grid=grid, in_specs=in_specs,
        out_specs=out_specs, interpret=_INTERPRET)


def _pad_rows(x, rows):
    return jnp.pad(x, ((0, rows - x.shape[0]),) + ((0, 0),) * (x.ndim - 1))


def kernel(world_pos, prev_world_pos, mesh_pos, node_type, cells, params):
    wp, pwp, mp = world_pos[0], prev_world_pos[0], mesh_pos[0]
    nt, cl = node_type[0], cells[0]
    n = wp.shape[0]
    n_tri = cl.shape[0]
    e = 6 * n_tri
    dummy = n

    n_pad = ((n + 1 + BR - 1) // BR) * BR
    grain = 2 * NW * CHUNK_G * NBUF_G
    e_pad = ((e + grain - 1) // grain) * grain
    kg = e_pad // (NW * CHUNK_G)
    ks = e_pad // (NW * CHUNK_S)
    assert e_pad % (NW * CHUNK_S * 2) == 0

    c0, c1, c2 = cl[:, 0], cl[:, 1], cl[:, 2]
    s_half = jnp.concatenate([c0, c1, c2])
    r_half = jnp.concatenate([c1, c2, c0])
    senders = jnp.concatenate([s_half, r_half]).astype(jnp.int32)
    receivers = jnp.concatenate([r_half, s_half]).astype(jnp.int32)
    pad_cnt = e_pad - e
    sidx = jnp.concatenate([senders, jnp.full((pad_cnt,), dummy, jnp.int32)])
    ridx = jnp.concatenate([receivers, jnp.full((pad_cnt,), dummy, jnp.int32)])
    sidx3 = sidx.reshape(NW, kg, CHUNK_G)
    ridx3 = ridx.reshape(NW, kg, CHUNK_G)
    ridx3_sc = ridx.reshape(NW, ks, CHUNK_S)

    pp8 = _pad_rows(jnp.concatenate(
        [wp, mp, jnp.zeros((n, 3), jnp.float32)], axis=1), n_pad)
    ppp8 = _pad_rows(jnp.concatenate(
        [pwp, mp, jnp.zeros((n, 3), jnp.float32)], axis=1), n_pad)
    nt_pad = _pad_rows(nt.astype(jnp.int32), n_pad)

    p = params
    nmean, nstd = p['node_norm_mean'], p['node_norm_std']
    emean, estd = p['edge_norm_mean'], p['edge_norm_std']
    omean, ostd = p['out_norm_mean'], p['out_norm_std']

    (ne1, ne1b), (ne2, ne2b), (ne3, ne3b) = p['node_enc']
    ne1f = ne1 / nstd[:, None]
    ne1bf = (ne1b - (nmean / nstd) @ ne1).reshape(1, LAT)
    w1v = jnp.zeros((8, LAT), jnp.float32).at[0:3].set(ne1f[0:3])
    w1o = jnp.zeros((16, LAT), jnp.float32).at[0:9].set(ne1f[3:12])

    (ee1, ee1b), (ee2, ee2b), (ee3, ee3b) = p['edge_enc']
    ee1f = ee1 / estd[:, None]
    ee1bf = (ee1b - (emean / estd) @ ee1).reshape(1, LAT)
    w1d = jnp.zeros((8, LAT), jnp.float32)
    w1d = w1d.at[0:3].set(ee1f[0:3]).at[3:5].set(ee1f[4:6])
    wnw = ee1f[3:4]
    wnm = ee1f[6:7]

    (de1, de1b), (de2, de2b), (de3, de3b) = p['dec']
    de3f = jnp.zeros((LAT, 8), jnp.float32).at[:, 0:3].set(de3 * ostd[None, :])
    de3bf = jnp.zeros((1, 8), jnp.float32).at[0, 0:3].set(de3b * ostd + omean)

    def split_edge_w1(blk):
        w1 = blk['edge'][0][0]
        return w1[0:LAT], w1[LAT:2 * LAT], w1[2 * LAT:3 * LAT]

    ws0, wr0, _ = split_edge_w1(p['proc'][0])

    grid_n = (n_pad // BR,)
    grid_e = (e_pad // BR,)
    wspec = _full((LAT, LAT))
    bspec = _full((1, LAT))
    sh_n = jax.ShapeDtypeStruct((n_pad, LAT), jnp.float32)
    sh_e = jax.ShapeDtypeStruct((e_pad, LAT), jnp.float32)

    nodes, tabs, tabr = _tc(
        _node_enc_body, [sh_n, sh_n, sh_n], grid_n,
        [_rb(8), _rb(8), _rb(1), _full((8, LAT)), _full((16, LAT)), bspec,
         wspec, bspec, wspec, bspec, wspec, wspec],
        [_rb(LAT)] * 3,
    )(pp8, ppp8, nt_pad, w1v, w1o, ne1bf, ne2, ne2b.reshape(1, LAT),
      ne3, ne3b.reshape(1, LAT), ws0, wr0)

    gs0, gr0, gsp, grp = _sc_gather2(tabs, tabr, sidx3, ridx3, pos_table=pp8)
    edges = _tc(
        _edge_enc_body, sh_e, grid_e,
        [_rb(8), _rb(8), _full((8, LAT)), bspec, bspec, bspec,
         wspec, bspec, wspec, bspec],
        _rb(LAT),
    )(gsp, grp, w1d, wnw, wnm, ee1bf, ee2, ee2b.reshape(1, LAT),
      ee3, ee3b.reshape(1, LAT))

    n_steps = len(p['proc'])
    for k in range(n_steps):
        blk = p['proc'][k]
        _, _, we = split_edge_w1(blk)
        (_, eb1), (ew2, eb2), (ew3, eb3) = blk['edge']
        nw1 = blk['node'][0][0]
        wn, wa = nw1[0:LAT], nw1[LAT:2 * LAT]
        (_, nb1), (nw2, nb2), (nw3, nb3) = blk['node']

        if k == 0:
            gs, gr = gs0, gr0
        else:
            gs, gr = _sc_gather2(tabs, tabr, sidx3, ridx3)
        edges = _tc(
            _edge_step_body, sh_e, grid_e,
            [_rb(LAT), _rb(LAT), _rb(LAT), wspec, bspec, wspec, bspec,
             wspec, bspec],
            _rb(LAT),
        )(gs, gr, edges, we, eb1.reshape(1, LAT), ew2, eb2.reshape(1, LAT),
          ew3, eb3.reshape(1, LAT))

        partials = _sc_scatter(edges, ridx3_sc, n_pad)
        ppspec = pl.BlockSpec((NC, BR, LAT), lambda i: (0, i, 0))
        if k + 1 < n_steps:
            wsn, wrn, _ = split_edge_w1(p['proc'][k + 1])
            nodes, tabs, tabr = _tc(
                _node_step_body, [sh_n, sh_n, sh_n], grid_n,
                [_rb(LAT), ppspec, wspec, wspec, bspec, wspec, bspec,
                 wspec, bspec, wspec, wspec],
                [_rb(LAT)] * 3,
            )(nodes, partials, wn, wa, nb1.reshape(1, LAT), nw2,
              nb2.reshape(1, LAT), nw3, nb3.reshape(1, LAT), wsn, wrn)
        else:
            nodes = _tc(
                _node_step_last_body, sh_n, grid_n,
                [_rb(LAT), ppspec, wspec, wspec, bspec, wspec, bspec,
                 wspec, bspec],
                _rb(LAT),
            )(nodes, partials, wn, wa, nb1.reshape(1, LAT), nw2,
              nb2.reshape(1, LAT), nw3, nb3.reshape(1, LAT))

    pos8 = _tc(
        _dec_body, jax.ShapeDtypeStruct((n_pad, 8), jnp.float32), grid_n,
        [_rb(LAT), _rb(8), _rb(8), wspec, bspec, wspec, bspec,
         _full((LAT, 8)), _full((1, 8))],
        _rb(8),
    )(nodes, pp8, ppp8, de1, de1b.reshape(1, LAT), de2,
      de2b.reshape(1, LAT), de3f, de3bf)

    return pos8[:n, 0:3][None]


# speedup vs baseline: 1.0239x; 1.0239x over previous
import functools

import jax
import jax.numpy as jnp
from jax import lax
from jax.experimental import pallas as pl
from jax.experimental.pallas import tpu as pltpu
from jax.experimental.pallas import tpu_sc as plsc

_INTERPRET = False

NC, NS = 2, 16
NW = NC * NS
CHUNK_G = 96
NBUF_G = 4
CHUNK_S = 128
LAT = 128
BR = 1024


def _ln_f(x):
    m = jnp.mean(x, axis=-1, keepdims=True)
    xc = x - m
    v = jnp.mean(xc * xc, axis=-1, keepdims=True)
    return xc * lax.rsqrt(v + 1e-5)


def _relu(x):
    return jnp.maximum(x, 0.0)


def _dot(a, b):
    return jnp.dot(a, b, preferred_element_type=jnp.float32)


def _rb(d):
    return pl.BlockSpec((BR, d), lambda i: (i, 0))


def _full(shape):
    return pl.BlockSpec(shape, lambda i: tuple(0 for _ in shape))


def _sc_gather2(table_s, table_r, sidx3, ridx3, pos_table=None):
    k_chunks = sidx3.shape[1]
    d = table_s.shape[1]
    e_pad = NW * k_chunks * CHUNK_G
    assert k_chunks % NBUF_G == 0
    mesh = plsc.VectorSubcoreMesh(core_axis_name="c", subcore_axis_name="s")
    dp = 0 if pos_table is None else pos_table.shape[1]
    out_type = [jax.ShapeDtypeStruct((e_pad, d), jnp.float32)] * 2
    pos_scratch = []
    pos_args = ()
    if pos_table is not None:
        out_type += [jax.ShapeDtypeStruct((e_pad, dp), jnp.float32)] * 2
        pos_scratch = [pltpu.VMEM((CHUNK_G, dp), jnp.float32)] * (2 * NBUF_G)
        pos_args = (pos_table,)

    @functools.partial(
        pl.kernel,
        out_type=out_type,
        mesh=mesh,
        compiler_params=pltpu.CompilerParams(
            use_tc_tiling_on_sc=False,
            disable_bounds_checks=True,
            disable_semaphore_checks=True),
        scratch_types=[
            pltpu.VMEM((k_chunks, CHUNK_G), jnp.int32),
            pltpu.VMEM((k_chunks, CHUNK_G), jnp.int32),
        ] + [pltpu.VMEM((CHUNK_G, d), jnp.float32)] * (2 * NBUF_G)
          + pos_scratch + [
            pltpu.SemaphoreType.DMA,
            pltpu.SemaphoreType.DMA,
        ],
    )
    def gather_kernel(ts, tr, *rest):
        if dp:
            (tp, si, ri, gs, gr, gsp, grp, siv, riv,
             s0, s1, s2, s3, r0, r1, r2, r3,
             p0, p1, p2, p3, q0, q1, q2, q3, gsem, wsem) = rest
        else:
            (si, ri, gs, gr, siv, riv,
             s0, s1, s2, s3, r0, r1, r2, r3, gsem, wsem) = rest
        w = lax.axis_index("s") * NC + lax.axis_index("c")
        pltpu.sync_copy(si.at[w], siv)
        pltpu.sync_copy(ri.at[w], riv)

        def run_phase(ta, tb, oa, ob, abuf, bbuf):
            def start(j, b):
                pltpu.async_copy(ta.at[siv.at[j]], abuf[b], gsem)
                pltpu.async_copy(tb.at[riv.at[j]], bbuf[b], gsem)

            def wait_gather(b):
                pltpu.make_async_copy(ta.at[siv.at[0]], abuf[b], gsem).wait()
                pltpu.make_async_copy(tb.at[riv.at[0]], bbuf[b], gsem).wait()

            def start_write(j, b):
                base = (w * k_chunks + j) * CHUNK_G
                pltpu.async_copy(abuf[b], oa.at[pl.ds(base, CHUNK_G)], wsem)
                pltpu.async_copy(bbuf[b], ob.at[pl.ds(base, CHUNK_G)], wsem)

            def wait_write(b):
                pltpu.make_async_copy(
                    abuf[b], oa.at[pl.ds(0, CHUNK_G)], wsem).wait()
                pltpu.make_async_copy(
                    bbuf[b], ob.at[pl.ds(0, CHUNK_G)], wsem).wait()

            for b in range(NBUF_G - 1):
                start(b, b)

            def body(g, carry):
                for b in range(NBUF_G):
                    j = g * NBUF_G + b
                    jn = j + NBUF_G - 1
                    bn = (b + NBUF_G - 1) % NBUF_G

                    @pl.when(jnp.logical_and(jn < k_chunks, j >= 1))
                    def _():
                        wait_write(bn)

                    @pl.when(jn < k_chunks)
                    def _():
                        start(jn, bn)

                    wait_gather(b)
                    start_write(j, b)
                return carry

            lax.fori_loop(0, k_chunks // NBUF_G, body, 0)
            for b in range(NBUF_G):
                wait_write(b)

        run_phase(ts, tr, gs, gr, [s0, s1, s2, s3], [r0, r1, r2, r3])
        if dp:
            run_phase(tp, tp, gsp, grp, [p0, p1, p2, p3], [q0, q1, q2, q3])

    return gather_kernel(table_s, table_r, *pos_args, sidx3, ridx3)


def _sc_scatter(vals, ridx3, n_acc):
    k_chunks = ridx3.shape[1]
    d = vals.shape[1]
    rows_tile = n_acc // NS
    mesh = plsc.VectorSubcoreMesh(core_axis_name="c", subcore_axis_name="s")
    zeros_tile = jnp.zeros((rows_tile, d), jnp.float32)

    @functools.partial(
        pl.kernel,
        out_type=jax.ShapeDtypeStruct((NC, n_acc, d), jnp.float32),
        mesh=mesh,
        compiler_params=pltpu.CompilerParams(
            disable_bounds_checks=True,
            disable_semaphore_checks=True),
        scratch_types=[
            pltpu.VMEM((k_chunks, CHUNK_S), jnp.int32),
            pltpu.VMEM((CHUNK_S, d), jnp.float32),
            pltpu.VMEM((CHUNK_S, d), jnp.float32),
            pltpu.VMEM_SHARED((n_acc, d), jnp.float32),
            pltpu.SemaphoreType.DMA,
        ],
    )
    def scatter_kernel(vals_r, ri, zrows, out, idxv, v0, v1, acc, lsem):
        c = lax.axis_index("c")
        s = lax.axis_index("s")
        w = s * NC + c
        pltpu.sync_copy(zrows, acc.at[pl.ds(s * rows_tile, rows_tile)])
        pltpu.sync_copy(ri.at[w], idxv)
        plsc.subcore_barrier()
        vbuf = [v0, v1]

        def startv(j, b):
            base = (w * k_chunks + j) * CHUNK_S
            pltpu.async_copy(vals_r.at[pl.ds(base, CHUNK_S)], vbuf[b], lsem)

        def waitv(b):
            pltpu.make_async_copy(vals_r.at[pl.ds(0, CHUNK_S)], vbuf[b],
                                  lsem).wait()

        startv(0, 0)

        def body(g, carry):
            for b in range(2):
                j = 2 * g + b

                @pl.when(j + 1 < k_chunks)
                def _():
                    startv(j + 1, 1 - b)

                waitv(b)
                pltpu.sync_copy(vbuf[b], acc.at[idxv.at[j]], add=True)
            return carry

        lax.fori_loop(0, k_chunks // 2, body, 0)
        plsc.subcore_barrier()
        pltpu.sync_copy(acc.at[pl.ds(s * rows_tile, rows_tile)],
                        out.at[c, pl.ds(s * rows_tile, rows_tile)])

    return scatter_kernel(vals, ridx3, zeros_tile)


def _node_enc_body(pp, ppp, nt, w1v, w1o, b1, w2, b2, w3, b3, ws0, wr0,
                   on, ots, otr):
    vel = pp[...] - ppp[...]
    oh = (lax.broadcasted_iota(jnp.int32, (vel.shape[0], 16), 1)
          == nt[...]).astype(jnp.float32)
    x = _relu(_dot(vel, w1v[...]) + _dot(oh, w1o[...]) + b1[...])
    x = _relu(_dot(x, w2[...]) + b2[...])
    y = _ln_f(_dot(x, w3[...]) + b3[...])
    on[...] = y
    ots[...] = _dot(y, ws0[...])
    otr[...] = _dot(y, wr0[...])


def _edge_enc_body(ps, pr, w1d, wnw, wnm, b1, w2, b2, w3, b3, out):
    dd = ps[...] - pr[...]
    dw = dd[:, 0:3]
    dm = dd[:, 3:5]
    nw = jnp.sqrt(jnp.sum(dw * dw, axis=-1, keepdims=True) + 1e-12)
    nm = jnp.sqrt(jnp.sum(dm * dm, axis=-1, keepdims=True) + 1e-12)
    x = _relu(_dot(dd, w1d[...]) + nw * wnw[...] + nm * wnm[...] + b1[...])
    x = _relu(_dot(x, w2[...]) + b2[...])
    out[...] = _ln_f(_dot(x, w3[...]) + b3[...])


def _edge_step_body(gs, gr, e, we, b1, w2, b2, w3, b3, out):
    er = e[...]
    x = _relu(gs[...] + gr[...] + _dot(er, we[...]) + b1[...])
    x = _relu(_dot(x, w2[...]) + b2[...])
    out[...] = _ln_f(_dot(x, w3[...]) + b3[...]) + er


def _node_step_body(n, pp, wn, wa, b1, w2, b2, w3, b3, wsn, wrn,
                    on, ots, otr):
    nr = n[...]
    p = pp[...]
    agg = p[0] + p[1]
    x = _relu(_dot(nr, wn[...]) + _dot(agg, wa[...]) + b1[...])
    x = _relu(_dot(x, w2[...]) + b2[...])
    y = _ln_f(_dot(x, w3[...]) + b3[...]) + nr
    on[...] = y
    ots[...] = _dot(y, wsn[...])
    otr[...] = _dot(y, wrn[...])


def _node_step_last_body(n, pp, wn, wa, b1, w2, b2, w3, b3, on):
    nr = n[...]
    p = pp[...]
    agg = p[0] + p[1]
    x = _relu(_dot(nr, wn[...]) + _dot(agg, wa[...]) + b1[...])
    x = _relu(_dot(x, w2[...]) + b2[...])
    on[...] = _ln_f(_dot(x, w3[...]) + b3[...]) + nr


def _dec_body(n, pp, ppp, w1, b1, w2, b2, w3, b3, out):
    x = _relu(_dot(n[...], w1[...]) + b1[...])
    x = _relu(_dot(x, w2[...]) + b2[...])
    x = _dot(x, w3[...]) + b3[...]
    out[...] = x + 2.0 * pp[...] - ppp[...]


def _tc(body, out_shapes, grid, in_specs, out_specs):
    return pl.pallas_call(
        body, out_shape=out_shapes, grid=grid, in_specs=in_specs,
        out_specs=out_specs, interpret=_INTERPRET)


def _pad_rows(x, rows):
    return jnp.pad(x, ((0, rows - x.shape[0]),) + ((0, 0),) * (x.ndim - 1))


def kernel(world_pos, prev_world_pos, mesh_pos, node_type, cells, params):
    wp, pwp, mp = world_pos[0], prev_world_pos[0], mesh_pos[0]
    nt, cl = node_type[0], cells[0]
    n = wp.shape[0]
    n_tri = cl.shape[0]
    e = 6 * n_tri
    dummy = n

    n_pad = ((n + 1 + BR - 1) // BR) * BR
    grain = 2 * NW * CHUNK_G * NBUF_G
    e_pad = ((e + grain - 1) // grain) * grain
    kg = e_pad // (NW * CHUNK_G)
    ks = e_pad // (NW * CHUNK_S)
    assert e_pad % (NW * CHUNK_S * 2) == 0

    c0, c1, c2 = cl[:, 0], cl[:, 1], cl[:, 2]
    s_half = jnp.concatenate([c0, c1, c2])
    r_half = jnp.concatenate([c1, c2, c0])
    senders = jnp.concatenate([s_half, r_half]).astype(jnp.int32)
    receivers = jnp.concatenate([r_half, s_half]).astype(jnp.int32)
    pad_cnt = e_pad - e
    sidx = jnp.concatenate([senders, jnp.full((pad_cnt,), dummy, jnp.int32)])
    ridx = jnp.concatenate([receivers, jnp.full((pad_cnt,), dummy, jnp.int32)])
    sidx3 = sidx.reshape(NW, kg, CHUNK_G)
    ridx3 = ridx.reshape(NW, kg, CHUNK_G)
    ridx3_sc = ridx.reshape(NW, ks, CHUNK_S)

    pp8 = _pad_rows(jnp.concatenate(
        [wp, mp, jnp.zeros((n, 3), jnp.float32)], axis=1), n_pad)
    ppp8 = _pad_rows(jnp.concatenate(
        [pwp, mp, jnp.zeros((n, 3), jnp.float32)], axis=1), n_pad)
    nt_pad = _pad_rows(nt.astype(jnp.int32), n_pad)

    p = params
    nmean, nstd = p['node_norm_mean'], p['node_norm_std']
    emean, estd = p['edge_norm_mean'], p['edge_norm_std']
    omean, ostd = p['out_norm_mean'], p['out_norm_std']

    (ne1, ne1b), (ne2, ne2b), (ne3, ne3b) = p['node_enc']
    ne1f = ne1 / nstd[:, None]
    ne1bf = (ne1b - (nmean / nstd) @ ne1).reshape(1, LAT)
    w1v = jnp.zeros((8, LAT), jnp.float32).at[0:3].set(ne1f[0:3])
    w1o = jnp.zeros((16, LAT), jnp.float32).at[0:9].set(ne1f[3:12])

    (ee1, ee1b), (ee2, ee2b), (ee3, ee3b) = p['edge_enc']
    ee1f = ee1 / estd[:, None]
    ee1bf = (ee1b - (emean / estd) @ ee1).reshape(1, LAT)
    w1d = jnp.zeros((8, LAT), jnp.float32)
    w1d = w1d.at[0:3].set(ee1f[0:3]).at[3:5].set(ee1f[4:6])
    wnw = ee1f[3:4]
    wnm = ee1f[6:7]

    (de1, de1b), (de2, de2b), (de3, de3b) = p['dec']
    de3f = jnp.zeros((LAT, 8), jnp.float32).at[:, 0:3].set(de3 * ostd[None, :])
    de3bf = jnp.zeros((1, 8), jnp.float32).at[0, 0:3].set(de3b * ostd + omean)

    def split_edge_w1(blk):
        w1 = blk['edge'][0][0]
        return w1[0:LAT], w1[LAT:2 * LAT], w1[2 * LAT:3 * LAT]

    ws0, wr0, _ = split_edge_w1(p['proc'][0])

    grid_n = (n_pad // BR,)
    grid_e = (e_pad // BR,)
    wspec = _full((LAT, LAT))
    bspec = _full((1, LAT))
    sh_n = jax.ShapeDtypeStruct((n_pad, LAT), jnp.float32)
    sh_e = jax.ShapeDtypeStruct((e_pad, LAT), jnp.float32)

    nodes, tabs, tabr = _tc(
        _node_enc_body, [sh_n, sh_n, sh_n], grid_n,
        [_rb(8), _rb(8), _rb(1), _full((8, LAT)), _full((16, LAT)), bspec,
         wspec, bspec, wspec, bspec, wspec, wspec],
        [_rb(LAT)] * 3,
    )(pp8, ppp8, nt_pad, w1v, w1o, ne1bf, ne2, ne2b.reshape(1, LAT),
      ne3, ne3b.reshape(1, LAT), ws0, wr0)

    gsp, grp = _sc_gather2(pp8, pp8, sidx3, ridx3)
    edges = _tc(
        _edge_enc_body, sh_e, grid_e,
        [_rb(8), _rb(8), _full((8, LAT)), bspec, bspec, bspec,
         wspec, bspec, wspec, bspec],
        _rb(LAT),
    )(gsp, grp, w1d, wnw, wnm, ee1bf, ee2, ee2b.reshape(1, LAT),
      ee3, ee3b.reshape(1, LAT))

    n_steps = len(p['proc'])
    for k in range(n_steps):
        blk = p['proc'][k]
        _, _, we = split_edge_w1(blk)
        (_, eb1), (ew2, eb2), (ew3, eb3) = blk['edge']
        nw1 = blk['node'][0][0]
        wn, wa = nw1[0:LAT], nw1[LAT:2 * LAT]
        (_, nb1), (nw2, nb2), (nw3, nb3) = blk['node']

        gs, gr = _sc_gather2(tabs, tabr, sidx3, ridx3)
        edges = _tc(
            _edge_step_body, sh_e, grid_e,
            [_rb(LAT), _rb(LAT), _rb(LAT), wspec, bspec, wspec, bspec,
             wspec, bspec],
            _rb(LAT),
        )(gs, gr, edges, we, eb1.reshape(1, LAT), ew2, eb2.reshape(1, LAT),
          ew3, eb3.reshape(1, LAT))

        partials = _sc_scatter(edges, ridx3_sc, n_pad)
        ppspec = pl.BlockSpec((NC, BR, LAT), lambda i: (0, i, 0))
        if k + 1 < n_steps:
            wsn, wrn, _ = split_edge_w1(p['proc'][k + 1])
            nodes, tabs, tabr = _tc(
                _node_step_body, [sh_n, sh_n, sh_n], grid_n,
                [_rb(LAT), ppspec, wspec, wspec, bspec, wspec, bspec,
                 wspec, bspec, wspec, wspec],
                [_rb(LAT)] * 3,
            )(nodes, partials, wn, wa, nb1.reshape(1, LAT), nw2,
              nb2.reshape(1, LAT), nw3, nb3.reshape(1, LAT), wsn, wrn)
        else:
            nodes = _tc(
                _node_step_last_body, sh_n, grid_n,
                [_rb(LAT), ppspec, wspec, wspec, bspec, wspec, bspec,
                 wspec, bspec],
                _rb(LAT),
            )(nodes, partials, wn, wa, nb1.reshape(1, LAT), nw2,
              nb2.reshape(1, LAT), nw3, nb3.reshape(1, LAT))

    pos8 = _tc(
        _dec_body, jax.ShapeDtypeStruct((n_pad, 8), jnp.float32), grid_n,
        [_rb(LAT), _rb(8), _rb(8), wspec, bspec, wspec, bspec,
         _full((LAT, 8)), _full((1, 8))],
        _rb(8),
    )(nodes, pp8, ppp8, de1, de1b.reshape(1, LAT), de2,
      de2b.reshape(1, LAT), de3f, de3bf)

    return pos8[:n, 0:3][None]
